# baseline (speedup 1.0000x reference)
import functools

import jax
import jax.numpy as jnp
from jax import lax
from jax.experimental import pallas as pl
from jax.experimental.pallas import tpu as pltpu
from jax.experimental.pallas import tpu_sc as plsc

_NC = 2
_NS = 16
_NW = _NC * _NS

_VS = 512
_OP = 136
_TAIL_V0 = 999936

_D = 64
_W = 128
_VOCAB = 1000000


def _mesh():
    return plsc.VectorSubcoreMesh(
        core_axis_name="c", subcore_axis_name="s", num_cores=_NC, num_subcores=_NS
    )


@jax.jit
def _format_table(table):
    tt = table.T

    @functools.partial(
        pl.kernel,
        out_type=jax.ShapeDtypeStruct((_VOCAB, _W), jnp.float32),
        mesh=_mesh(),
        scratch_types=[
            pltpu.VMEM((_D, _VS), jnp.float32),
            pltpu.VMEM((_D, _VS), jnp.float32),
            pltpu.VMEM((_VS // 4, _OP), jnp.float32),
            pltpu.SemaphoreType.DMA,
            pltpu.SemaphoreType.DMA,
        ],
        compiler_params=pltpu.CompilerParams(
            use_tc_tiling_on_sc=True, needs_layout_passes=False
        ),
    )
    def k1(tt_hbm, tail_hbm, o_hbm, sbuf0, sbuf1, obuf, sem0, sem1):
        wid = lax.axis_index("s") * _NC + lax.axis_index("c")

        def start_load(v0, sbuf, sem):
            pltpu.async_copy(tt_hbm.at[:, pl.ds(v0, _VS)], sbuf, sem)

        def wait_load(v0, sbuf, sem):
            pltpu.make_async_copy(tt_hbm.at[:, pl.ds(v0, _VS)], sbuf, sem).wait()

        iota = lax.iota(jnp.int32, 16)

        def emit_half(sbuf, v0, base):
            @plsc.parallel_loop(0, (_VS // 4 // 16) * 4, 1, unroll=1)
            def per_blk(i):
                pg = lax.div(i, 4)
                e0 = lax.rem(i, 4) * 16
                r0 = base + pg * 16
                for d in range(16):
                    dv = lax.rem(iota + d, 16)
                    v = plsc.load_gather(sbuf, [e0 + dv, r0 + iota])
                    plsc.store_scatter(obuf, [pg * 16 + iota, e0 + dv], v)

            pltpu.sync_copy(
                obuf.at[pl.ds(0, _VS // 4), pl.ds(0, _W)],
                o_hbm.at[pl.ds(v0 + base, _VS // 4)],
            )

        def slab_v0(t):
            return (wid + _NW * t) * _VS

        n_slabs = 61 + jnp.where(wid == 0, 1, 0)
        start_load(slab_v0(0), sbuf0, sem0)

        def body(t, carry):
            @pl.when(t + 1 < n_slabs)
            def _():
                @pl.when(lax.rem(t + 1, 2) == 0)
                def _():
                    start_load(slab_v0(t + 1), sbuf0, sem0)

                @pl.when(lax.rem(t + 1, 2) == 1)
                def _():
                    start_load(slab_v0(t + 1), sbuf1, sem1)

            v0 = slab_v0(t)

            @pl.when(lax.rem(t, 2) == 0)
            def _():
                wait_load(v0, sbuf0, sem0)
                for q in range(4):
                    emit_half(sbuf0, v0, q * (_VS // 4))

            @pl.when(lax.rem(t, 2) == 1)
            def _():
                wait_load(v0, sbuf1, sem1)
                for q in range(4):
                    emit_half(sbuf1, v0, q * (_VS // 4))

            return carry

        lax.fori_loop(0, n_slabs, body, 0)

        @pl.when(wid == 1)
        def _():
            for prel in range(64):
                pltpu.async_copy(
                    tail_hbm.at[pl.ds(prel * _D, _D)],
                    obuf.at[prel, pl.ds(0, _D)],
                    sem0,
                )
            for prel in range(64):
                pltpu.make_async_copy(
                    tail_hbm.at[pl.ds(prel * _D, _D)],
                    obuf.at[prel, pl.ds(0, _D)],
                    sem0,
                ).wait()
            pltpu.sync_copy(
                obuf.at[pl.ds(0, 64), pl.ds(0, _W)],
                o_hbm.at[pl.ds(_TAIL_V0, 64)],
            )

    tail = table[_TAIL_V0:].reshape(64 * _D)
    return k1(tt, tail)


@jax.jit
def _gather(x, t2):
    x4 = x.T.reshape(25, 8, 32, 128).transpose(0, 2, 1, 3)

    @functools.partial(
        pl.kernel,
        out_type=jax.ShapeDtypeStruct((200, 8, 32, 8, 128), jnp.float32),
        mesh=_mesh(),
        scratch_types=[
            pltpu.VMEM((25, 8, 128), jnp.int32),
            pltpu.VMEM((128, _W), jnp.float32),
            pltpu.VMEM((128, _W), jnp.float32),
            pltpu.VMEM((8, 8, _OP), jnp.float32),
            pltpu.SemaphoreType.DMA,
            pltpu.SemaphoreType.DMA,
            pltpu.SemaphoreType.DMA,
        ],
        compiler_params=pltpu.CompilerParams(needs_layout_passes=False),
    )
    def k2(x4_hbm, t_hbm, o_hbm, idxb, gbuf0, gbuf1, tbuf, isem, gsem0, gsem1):
        wid = lax.axis_index("s") * _NC + lax.axis_index("c")
        j = wid

        for sb in range(25):
            pltpu.async_copy(x4_hbm.at[sb, j], idxb.at[sb], isem)
        for sb in range(25):
            pltpu.make_async_copy(x4_hbm.at[sb, j], idxb.at[sb], isem).wait()

        def start_gather(u, gbuf, sem):
            pltpu.async_copy(t_hbm.at[idxb.at[u // 8, lax.rem(u, 8)]], gbuf, sem)

        def wait_gather(u, gbuf, sem):
            pltpu.make_async_copy(
                t_hbm.at[idxb.at[u // 8, lax.rem(u, 8)]], gbuf, sem
            ).wait()

        iota = lax.iota(jnp.int32, 16)

        def transpose_unit(gbuf):
            @plsc.parallel_loop(0, 128, 1, unroll=8)
            def per_l(l):
                lv = jnp.zeros((16,), jnp.int32) + l
                for k in range(4):
                    ev = iota + 16 * k
                    v = gbuf[l, pl.ds(16 * k, 16)]
                    plsc.store_scatter(
                        tbuf,
                        [lax.shift_right_logical(ev, 3), lax.rem(ev, 8), lv],
                        v,
                    )

        start_gather(0, gbuf0, gsem0)

        def body(u, carry):
            @pl.when(u + 1 < 200)
            def _():
                @pl.when(lax.rem(u + 1, 2) == 0)
                def _():
                    start_gather(u + 1, gbuf0, gsem0)

                @pl.when(lax.rem(u + 1, 2) == 1)
                def _():
                    start_gather(u + 1, gbuf1, gsem1)

            @pl.when(lax.rem(u, 2) == 0)
            def _():
                wait_gather(u, gbuf0, gsem0)
                transpose_unit(gbuf0)

            @pl.when(lax.rem(u, 2) == 1)
            def _():
                wait_gather(u, gbuf1, gsem1)
                transpose_unit(gbuf1)

            pltpu.sync_copy(
                tbuf.at[:, :, pl.ds(0, _W)],
                o_hbm.at[u, :, j],
            )
            return carry

        lax.fori_loop(0, 200, body, 0)

    return k2(x4, t2)


def kernel(x, table):
    t2 = _format_table(table)
    o = _gather(x, t2)
    out = o.transpose(2, 4, 0, 1, 3)
    return out.reshape(4096, 200, 64)


# speedup vs baseline: 1.3294x; 1.3294x over previous
import functools

import jax
import jax.numpy as jnp
from jax import lax
from jax.experimental import pallas as pl
from jax.experimental.pallas import tpu as pltpu
from jax.experimental.pallas import tpu_sc as plsc

_NC = 2
_NS = 16
_NW = _NC * _NS
_CHUNK = 800


@jax.jit
def _gather_rows(table, idx):
    n = idx.shape[0]
    d = table.shape[1]
    b_per_w = n // _NW
    n_chunks = b_per_w // _CHUNK
    mesh = plsc.VectorSubcoreMesh(
        core_axis_name="c", subcore_axis_name="s", num_cores=_NC, num_subcores=_NS
    )

    @functools.partial(
        pl.kernel,
        out_type=jax.ShapeDtypeStruct((n, d), jnp.float32),
        mesh=mesh,
        scratch_types=[
            pltpu.VMEM((b_per_w,), jnp.int32),
            pltpu.VMEM((2, _CHUNK, d), jnp.float32),
            pltpu.SemaphoreType.DMA,
            pltpu.SemaphoreType.DMA,
        ],
        compiler_params=pltpu.CompilerParams(use_tc_tiling_on_sc=False),
    )
    def k(table_hbm, idx_hbm, out_hbm, idx_v, rows_v, gsem0, gsem1):
        wid = lax.axis_index("s") * _NC + lax.axis_index("c")
        base = wid * b_per_w
        pltpu.sync_copy(idx_hbm.at[pl.ds(base, b_per_w)], idx_v)

        def start_gather(i, buf, sem):
            off = i * _CHUNK
            pltpu.async_copy(
                table_hbm.at[idx_v.at[pl.ds(off, _CHUNK)]], rows_v.at[buf], sem
            )

        def wait_gather(i, buf, sem):
            off = i * _CHUNK
            pltpu.make_async_copy(
                table_hbm.at[idx_v.at[pl.ds(off, _CHUNK)]], rows_v.at[buf], sem
            ).wait()

        start_gather(0, 0, gsem0)

        def body(i, carry):
            @pl.when(i + 1 < n_chunks)
            def _():
                @pl.when(lax.rem(i + 1, 2) == 0)
                def _():
                    start_gather(i + 1, 0, gsem0)

                @pl.when(lax.rem(i + 1, 2) == 1)
                def _():
                    start_gather(i + 1, 1, gsem1)

            @pl.when(lax.rem(i, 2) == 0)
            def _():
                wait_gather(i, 0, gsem0)
                pltpu.sync_copy(rows_v.at[0], out_hbm.at[pl.ds(base + i * _CHUNK, _CHUNK)])

            @pl.when(lax.rem(i, 2) == 1)
            def _():
                wait_gather(i, 1, gsem1)
                pltpu.sync_copy(rows_v.at[1], out_hbm.at[pl.ds(base + i * _CHUNK, _CHUNK)])

            return carry

        lax.fori_loop(0, n_chunks, body, 0)

    return k(table, idx)


def kernel(x, table):
    b, s = x.shape
    d = table.shape[1]
    out = _gather_rows(table, x.reshape(b * s))
    return out.reshape(b, s, d)
